# baseline (speedup 1.0000x reference)
import functools

import jax
import jax.numpy as jnp
import numpy as np
from jax import lax
from jax.experimental import pallas as pl
from jax.experimental.pallas import tpu as pltpu
from jax.experimental.pallas import tpu_sc as plsc

NUM_VOCAB = 1000000
D_MODEL = 64
BATCH = 4096
SEQ = 200
NUM_WORKERS = 32
BW = BATCH // NUM_WORKERS
SCALE = float(np.sqrt(float(D_MODEL)))


def _position_encoding(max_len, d_model):
    pe = np.zeros((max_len, d_model), dtype=np.float32)
    position = np.arange(0, max_len, dtype=np.float32)[:, None]
    div_term = np.exp(-np.arange(0, d_model, 2, dtype=np.float32)
                      * (np.log(10000.0) / d_model))
    pe[:, 0::2] = np.sin(position * div_term)
    pe[:, 1::2] = np.cos(position * div_term)
    return pe


_PE = _position_encoding(800, D_MODEL)[:SEQ, :].reshape(SEQ // 2, 128)

_mesh = plsc.VectorSubcoreMesh(core_axis_name="c", subcore_axis_name="s")


@functools.partial(
    pl.kernel,
    mesh=_mesh,
    out_type=jax.ShapeDtypeStruct((SEQ, D_MODEL, BATCH), jnp.float32),
    scratch_types=[
        pltpu.VMEM((SEQ, BW), jnp.int32),
        pltpu.VMEM((2, BW), jnp.int32),
        pltpu.VMEM((2, BW, 128), jnp.float32),
        pltpu.VMEM((2, D_MODEL, BW + 1), jnp.float32),
        pltpu.VMEM((SEQ // 2, 128), jnp.float32),
        pltpu.SemaphoreType.DMA,
        pltpu.SemaphoreType.DMA,
        pltpu.SemaphoreType.DMA,
        pltpu.SemaphoreType.DMA,
    ],
    compiler_params=pltpu.CompilerParams(use_tc_tiling_on_sc=True,
                                         needs_layout_passes=False),
)
def _emb_lookup(xt_hbm, tab_hbm, pe_hbm, out_hbm,
                idx_v, gidx, gbuf, sbuf, pe_v, gsem0, gsem1, osem0, osem1):
    wid = lax.axis_index("s") * 2 + lax.axis_index("c")
    gsems = (gsem0, gsem1)
    osems = (osem0, osem1)
    col0 = wid * BW

    pltpu.sync_copy(pe_hbm, pe_v)
    pltpu.sync_copy(xt_hbm.at[:, pl.ds(col0, BW)], idx_v)

    def prep_and_issue(p, b):
        for k in range(BW // 16):
            sl = pl.ds(k * 16, 16)
            gidx[b, sl] = lax.shift_right_logical(idx_v[p, sl], 1)
        pltpu.async_copy(tab_hbm.at[gidx.at[b]], gbuf.at[b], gsems[b])

    def drain_gather(b):
        pltpu.make_async_copy(tab_hbm.at[pl.ds(0, BW)], gbuf.at[b],
                              gsems[b]).wait()

    def drain_store(b):
        pltpu.make_async_copy(sbuf.at[b, :, pl.ds(0, BW)],
                              out_hbm.at[0, :, pl.ds(0, BW)],
                              osems[b]).wait()

    prep_and_issue(0, 0)

    lane = lax.iota(jnp.int32, 16)
    _SPLAT = [jnp.full((16,), i, jnp.int32) for i in range(16)]

    def step(k, carry):
        for b in range(2):
            p = k * 2 + b

            @pl.when(p + 1 < SEQ)
            def _():
                prep_and_issue(p + 1, 1 - b)

            drain_gather(b)

            @pl.when(p >= 2)
            def _():
                drain_store(b)

            pe_vecs = [pe_v[k, pl.ds(b * D_MODEL + g * 16, 16)]
                       for g in range(D_MODEL // 16)]

            @plsc.parallel_loop(0, BW // 16, unroll=2)
            def comp(kb):
                base = kb * 16
                idxv = idx_v[p, pl.ds(base, 16)]
                offv = lax.mul(lax.bitwise_and(idxv, 1), D_MODEL)
                for i in range(16):
                    bi = base + i
                    bcol = jnp.broadcast_to(bi, (16,))
                    offs = lax.gather(
                        offv, _SPLAT[i][:, None],
                        lax.GatherDimensionNumbers(
                            offset_dims=(), collapsed_slice_dims=(0,),
                            start_index_map=(0,)),
                        (1,), mode=lax.GatherScatterMode.PROMISE_IN_BOUNDS)
                    for g in range(D_MODEL // 16):
                        pe_vec = pe_vecs[g]
                        col = offs + (lane + g * 16)
                        val = plsc.load_gather(gbuf.at[b], [bcol, col])
                        res = val * SCALE + pe_vec
                        plsc.store_scatter(sbuf.at[b],
                                           [lane + (g * 16), bcol], res)

            pltpu.async_copy(sbuf.at[b, :, pl.ds(0, BW)],
                             out_hbm.at[p, :, pl.ds(col0, BW)], osems[b])
        return carry

    lax.fori_loop(0, SEQ // 2, step, 0)
    drain_store(0)
    drain_store(1)


def kernel(x, table):
    xt = x.T
    tab = table.reshape(NUM_VOCAB // 2, 128)
    pe = jnp.asarray(_PE)
    out_t = _emb_lookup(xt, tab, pe)
    return out_t.transpose(2, 0, 1)


# speedup vs baseline: 1.1853x; 1.1853x over previous
import functools

import jax
import jax.numpy as jnp
import numpy as np
from jax import lax
from jax.experimental import pallas as pl
from jax.experimental.pallas import tpu as pltpu
from jax.experimental.pallas import tpu_sc as plsc

NUM_VOCAB = 1000000
D_MODEL = 64
BATCH = 4096
SEQ = 200
NUM_WORKERS = 32
BW = BATCH // NUM_WORKERS
SCALE = float(np.sqrt(float(D_MODEL)))


def _position_encoding(max_len, d_model):
    pe = np.zeros((max_len, d_model), dtype=np.float32)
    position = np.arange(0, max_len, dtype=np.float32)[:, None]
    div_term = np.exp(-np.arange(0, d_model, 2, dtype=np.float32)
                      * (np.log(10000.0) / d_model))
    pe[:, 0::2] = np.sin(position * div_term)
    pe[:, 1::2] = np.cos(position * div_term)
    return pe


_PE = _position_encoding(800, D_MODEL)[:SEQ, :].reshape(SEQ // 2, 128)

_mesh = plsc.VectorSubcoreMesh(core_axis_name="c", subcore_axis_name="s")


@functools.partial(
    pl.kernel,
    mesh=_mesh,
    out_type=jax.ShapeDtypeStruct((SEQ, D_MODEL, BATCH), jnp.float32),
    scratch_types=[
        pltpu.VMEM((SEQ, BW), jnp.int32),
        pltpu.VMEM((2, BW), jnp.int32),
        pltpu.VMEM((2, BW, 128), jnp.float32),
        pltpu.VMEM((2, D_MODEL, BW + 1), jnp.float32),
        pltpu.VMEM((SEQ // 2, 128), jnp.float32),
        pltpu.SemaphoreType.DMA,
        pltpu.SemaphoreType.DMA,
        pltpu.SemaphoreType.DMA,
        pltpu.SemaphoreType.DMA,
    ],
    compiler_params=pltpu.CompilerParams(use_tc_tiling_on_sc=True,
                                         needs_layout_passes=False),
)
def _emb_lookup(xt_hbm, tab_hbm, pe_hbm, out_hbm,
                idx_v, gidx, gbuf, sbuf, pe_v, gsem0, gsem1, osem0, osem1):
    wid = lax.axis_index("s") * 2 + lax.axis_index("c")
    gsems = (gsem0, gsem1)
    osems = (osem0, osem1)
    col0 = wid * BW

    pltpu.sync_copy(pe_hbm, pe_v)
    pltpu.sync_copy(xt_hbm.at[:, pl.ds(col0, BW)], idx_v)

    def prep_and_issue(p, b):
        for k in range(BW // 16):
            sl = pl.ds(k * 16, 16)
            gidx[b, sl] = lax.shift_right_logical(idx_v[p, sl], 1)
        pltpu.async_copy(tab_hbm.at[gidx.at[b]], gbuf.at[b], gsems[b])

    def drain_gather(b):
        pltpu.make_async_copy(tab_hbm.at[pl.ds(0, BW)], gbuf.at[b],
                              gsems[b]).wait()

    def drain_store(b):
        pltpu.make_async_copy(sbuf.at[b, :, pl.ds(0, BW)],
                              out_hbm.at[0, :, pl.ds(0, BW)],
                              osems[b]).wait()

    prep_and_issue(0, 0)

    lane = lax.iota(jnp.int32, 16)
    _SPLAT = [jnp.full((16,), i, jnp.int32) for i in range(16)]

    def step(k, carry):
        for b in range(2):
            p = k * 2 + b

            @pl.when(p + 1 < SEQ)
            def _():
                prep_and_issue(p + 1, 1 - b)

            drain_gather(b)

            @pl.when(p >= 2)
            def _():
                drain_store(b)

            pe_vecs = [pe_v[k, pl.ds(b * D_MODEL + g * 16, 16)]
                       for g in range(D_MODEL // 16)]
            offv_l = [lax.mul(lax.bitwise_and(idx_v[p, pl.ds(kb * 16, 16)],
                                              1), D_MODEL)
                      for kb in range(BW // 16)]
            rowv_l = [lane + kb * 16 for kb in range(BW // 16)]
            dn = lax.GatherDimensionNumbers(
                offset_dims=(), collapsed_slice_dims=(0,),
                start_index_map=(0,))

            for g in range(D_MODEL // 16):
                pe_g = pe_vecs[g]

                @plsc.parallel_loop(0, 16)
                def comp(d16):
                    d = g * 16 + d16
                    ped = lax.gather(
                        pe_g, jnp.broadcast_to(d16, (16,))[:, None], dn,
                        (1,), mode=lax.GatherScatterMode.PROMISE_IN_BOUNDS)
                    for kb in range(BW // 16):
                        col = offv_l[kb] + d
                        val = plsc.load_gather(gbuf.at[b],
                                               [rowv_l[kb], col])
                        res = val * SCALE + ped
                        sbuf[b, d, pl.ds(kb * 16, 16)] = res

            pltpu.async_copy(sbuf.at[b, :, pl.ds(0, BW)],
                             out_hbm.at[p, :, pl.ds(col0, BW)], osems[b])
        return carry

    lax.fori_loop(0, SEQ // 2, step, 0)
    drain_store(0)
    drain_store(1)


def kernel(x, table):
    xt = x.T
    tab = table.reshape(NUM_VOCAB // 2, 128)
    pe = jnp.asarray(_PE)
    out_t = _emb_lookup(xt, tab, pe)
    return out_t.transpose(2, 0, 1)
